# baseline (speedup 1.0000x reference)
import functools

import jax
import jax.numpy as jnp
from jax import lax
from jax.experimental import pallas as pl
from jax.experimental.pallas import tpu as pltpu
from jax.experimental.pallas import tpu_sc as plsc

_ROW_BLK = 2000
_BIG_IDX = 2**30


def _argmin_body(x1_ref, x2_ref, out_ref, vacc, iacc, *, half_groups):
    j = pl.program_id(0)
    nb = pl.num_programs(0)
    n_groups = _ROW_BLK // 8
    n_lgrp = vacc.shape[1] // 128

    @pl.when(j == 0)
    def _init():
        vacc[...] = jnp.full(vacc.shape, jnp.inf, dtype=vacc.dtype)
        iacc[...] = jnp.zeros(iacc.shape, dtype=iacc.dtype)

    for l in range(n_lgrp):
        lanes = pl.ds(l * 128, 128)
        v = vacc[:, lanes]
        i = iacc[:, lanes]
        for g in range(n_groups):
            xg = x1_ref[0, pl.ds(g * 8, 8), lanes]
            cmp = xg < v
            v = jnp.minimum(v, xg)
            i = jnp.where(cmp, j * n_groups + g, i)
        for g in range(n_groups):
            xg = x2_ref[0, pl.ds(g * 8, 8), lanes]
            cmp = xg < v
            v = jnp.minimum(v, xg)
            i = jnp.where(cmp, half_groups + j * n_groups + g, i)
        vacc[:, lanes] = v
        iacc[:, lanes] = i

    @pl.when(j == nb - 1)
    def _merge():
        vf = vacc[...]
        sub = lax.broadcasted_iota(jnp.int32, vf.shape, 0)
        gidx = iacc[...] * 8 + sub
        gmin = jnp.min(vf, axis=0, keepdims=True)
        cand = jnp.where(vf == gmin, gidx, _BIG_IDX)
        out_ref[...] = jnp.min(cand, axis=0, keepdims=True)


def _argmin_cols(xt):
    n_protos, n_samples = xt.shape
    half = n_protos // 2
    nb = half // _ROW_BLK
    x3 = xt.reshape(2, half, n_samples)
    out = pl.pallas_call(
        functools.partial(_argmin_body, half_groups=half // 8),
        grid=(nb,),
        in_specs=[
            pl.BlockSpec((1, _ROW_BLK, n_samples), lambda j: (0, j, 0)),
            pl.BlockSpec((1, _ROW_BLK, n_samples), lambda j: (1, j, 0)),
        ],
        out_specs=pl.BlockSpec((1, n_samples), lambda j: (0, 0)),
        out_shape=jax.ShapeDtypeStruct((1, n_samples), jnp.int32),
        scratch_shapes=[
            pltpu.VMEM((8, n_samples), jnp.float32),
            pltpu.VMEM((8, n_samples), jnp.int32),
        ],
        compiler_params=pltpu.CompilerParams(
            dimension_semantics=("arbitrary",),
        ),
    )(x3, x3)
    return out.reshape(n_samples)


def _label_gather(labels, win_idx):
    info = plsc.get_sparse_core_info()
    n_workers = info.num_cores * info.num_subcores
    b = win_idx.shape[0]
    b_per_w = b // n_workers
    mesh = plsc.VectorSubcoreMesh(core_axis_name="c", subcore_axis_name="s")

    @functools.partial(
        pl.kernel,
        mesh=mesh,
        out_type=jax.ShapeDtypeStruct((b,), labels.dtype),
        scratch_types=[
            pltpu.VMEM((b_per_w,), jnp.int32),
            pltpu.VMEM((b_per_w,), jnp.int32),
            pltpu.SemaphoreType.DMA,
        ],
    )
    def gather_kernel(labels_hbm, idx_hbm, out_hbm, idx_v, out_v, sem):
        wid = lax.axis_index("s") * info.num_cores + lax.axis_index("c")
        base = wid * b_per_w
        pltpu.sync_copy(idx_hbm.at[pl.ds(base, b_per_w)], idx_v)
        pltpu.async_copy(labels_hbm.at[idx_v], out_v, sem).wait()
        pltpu.sync_copy(out_v, out_hbm.at[pl.ds(base, b_per_w)])

    return gather_kernel(labels, win_idx)


def kernel(distances, labels):
    win_idx = _argmin_cols(distances.T)
    return _label_gather(labels, win_idx)
